# Initial kernel scaffold; baseline (speedup 1.0000x reference)
import jax
import jax.numpy as jnp
from jax.experimental import pallas as pl


def kernel(kernel_offsets, kernel_amplitudes, kernel_sharpness, seq_len):
    raise NotImplementedError("write your pallas kernel here")


# speedup vs baseline: 2110.9422x; 2110.9422x over previous
import jax
import jax.numpy as jnp
from jax import lax
from jax.experimental import pallas as pl
from jax.experimental.pallas import tpu as pltpu
from jax.experimental.pallas import tpu_sc as plsc

NK = 5
NH = 12
S = 2048
NC = 2
NS = 16
NW = NC * NS
ROWS_W = S // NW
WIN = 2560


def _body(p_hbm, out_hbm, pv, rloc, sem):
    c = lax.axis_index("c")
    s = lax.axis_index("s")
    w = s * NC + c
    pltpu.sync_copy(p_hbm, pv)

    iota = lax.iota(jnp.int32, 16).astype(jnp.float32)
    p0 = w % 8
    m0 = 64 * (w // 8)
    bw = 1545 - p0 - 8 * m0
    bwf = bw.astype(jnp.float32)

    for h in range(NH):
        hrow = jnp.full((16,), h, jnp.int32)

        def splat(j):
            return plsc.load_gather(pv, [hrow, jnp.full((16,), j, jnp.int32)])

        co = [splat(1 + n) + bwf for n in range(NK)]
        ns = [-jnp.abs(splat(1 + NK + n)) for n in range(NK)]
        am = [splat(1 + 2 * NK + n) for n in range(NK)]

        @pl.loop(0, WIN // 16)
        def _compute(v):
            t0 = pl.multiple_of(v * 16, 16)
            tvec = iota + t0.astype(jnp.float32)
            acc = jnp.zeros((16,), jnp.float32)
            for n in range(NK):
                d = co[n] + tvec
                acc = acc + am[n] * jnp.exp(ns[n] * (d * d))
            rloc[h, pl.ds(t0, 16)] = acc

        if h > 0:
            blk = out_hbm.at[h - 1, pl.ds(0, ROWS_W)]
            pltpu.make_async_copy(blk, blk, sem).wait()

        @pl.loop(0, ROWS_W)
        def _fire(m):
            i = 8 * (m0 + m) + p0
            start = pl.multiple_of(8 * (63 - m), 8)
            pltpu.async_copy(rloc.at[h, pl.ds(start, S)], out_hbm.at[h, i], sem)

    blk = out_hbm.at[NH - 1, pl.ds(0, ROWS_W)]
    pltpu.make_async_copy(blk, blk, sem).wait()


def _build():
    mesh = plsc.VectorSubcoreMesh(core_axis_name="c", subcore_axis_name="s")
    return pl.kernel(
        _body,
        out_type=jax.ShapeDtypeStruct((NH, S, S), jnp.float32),
        mesh=mesh,
        scratch_types=[
            pltpu.VMEM((NH, 16), jnp.float32),
            pltpu.VMEM((NH, WIN), jnp.float32),
            pltpu.SemaphoreType.DMA,
        ],
        compiler_params=pltpu.CompilerParams(
            use_tc_tiling_on_sc=False, needs_layout_passes=False
        ),
    )


def kernel(kernel_offsets, kernel_amplitudes, kernel_sharpness, seq_len):
    zf = jnp.asarray(seq_len, jnp.float32) - jnp.float32(S)
    offz = kernel_offsets.astype(jnp.float32) - zf - jnp.float32(2048.0)
    p = jnp.zeros((NH, 16), jnp.float32)
    p = p.at[:, 1:1 + NK].set(offz.T)
    p = p.at[:, 1 + NK:1 + 2 * NK].set(kernel_sharpness.astype(jnp.float32).T)
    p = p.at[:, 1 + 2 * NK:16].set(kernel_amplitudes.astype(jnp.float32).T)
    return _build()(p)
